# baseline (device time: 343404 ns/iter reference)
import jax
import jax.numpy as jnp
from jax import lax
from jax.experimental import pallas as pl
from jax.experimental.pallas import tpu as pltpu

M = 4096
N = 4096
K = 8192

P = 8
SM = M // P
BK = 256
NQ = 4
WQ = N // NQ
WH = N // 2

ROUNDS = 4
CA = 1280
CO = N - 2 * CA
CL = CA + CO
CH = CL // 2


def _ring_coords(q, my_y):
    xq = q // 4
    zq = jnp.where(xq == 0, q, 7 - q)
    return (xq, my_y, zq)


def _body(
    p_ref,
    dy_ref,
    w_ref,
    out_ref,
    acc,
    dy16,
    gather,
    stage,
    y_send, y_recv,
    fwd_send, fwd_recv,
    bwd_send, bwd_recv,
    rung_send, rung_recv,
    copy_sem,
):
    n = pl.program_id(0)
    k = pl.program_id(1)
    n_last = (n == pl.num_programs(0) - 1) & (k == pl.num_programs(1) - 1)

    my_x = lax.axis_index("x")
    my_y = lax.axis_index("y")
    my_z = lax.axis_index("z")
    p = p_ref[0]
    right = _ring_coords((p + 1) % P, my_y)
    left = _ring_coords((p - 1) % P, my_y)
    ynbr = (my_x, 1 - my_y, my_z)
    anti = (p + 4) % P

    @pl.when((n == 0) & (k == 0))
    def _():
        barrier = pltpu.get_barrier_semaphore()
        for nbr in (ynbr, left, right):
            pl.semaphore_signal(
                barrier, inc=1, device_id=nbr,
                device_id_type=pltpu.DeviceIdType.MESH,
            )
        pl.semaphore_wait(barrier, 3)

    @pl.when(k == 0)
    def _():
        acc[...] = jnp.zeros_like(acc)

    b = w_ref[...].astype(jnp.bfloat16)

    @pl.when(n == 0)
    def _():
        a = dy_ref[...].astype(jnp.bfloat16)
        dy16[:, pl.ds(k * BK, BK)] = a
        acc[...] += lax.dot_general(
            a, b, (((1,), (1,)), ((), ())),
            preferred_element_type=jnp.float32,
        )

    @pl.when(n != 0)
    def _():
        a = dy16[:, pl.ds(k * BK, BK)]
        acc[...] += lax.dot_general(
            a, b, (((1,), (1,)), ((), ())),
            preferred_element_type=jnp.float32,
        )

    def _y_quarter(h):
        return pltpu.make_async_remote_copy(
            src_ref=gather.at[p, :, pl.ds(h * WQ, WQ)],
            dst_ref=gather.at[anti, :, pl.ds(h * WQ, WQ)],
            send_sem=y_send.at[h], recv_sem=y_recv.at[h],
            device_id=ynbr, device_id_type=pltpu.DeviceIdType.MESH,
        )

    for h in range(NQ):
        @pl.when((n == h) & (k == pl.num_programs(1) - 1))
        def _(h=h):
            gather[p, :, pl.ds(h * WQ, WQ)] = acc[...].astype(jnp.bfloat16)
            _y_quarter(h).start()

    @pl.when(n_last)
    def _():
        for h in range(NQ):
            _y_quarter(h).wait()
        gather[p] = (
            gather[p].astype(jnp.float32) + gather[anti].astype(jnp.float32)
        ).astype(jnp.bfloat16)

        for yv in (0, 1):

            @pl.when(my_y == yv)
            def _(yv=yv):
                ro = CA * yv
                rs = CL * yv

                half_copies = [None, None]

                def _emit(s):
                    for h in (0, 1):
                        if half_copies[h] is not None:
                            half_copies[h].wait()
                        stage[h] = gather[
                            s, :, h * WH:(h + 1) * WH
                        ].astype(jnp.float32)
                        cp = pltpu.make_async_copy(
                            stage.at[h],
                            out_ref.at[pl.ds(s * SM, SM), pl.ds(h * WH, WH)],
                            copy_sem.at[h],
                        )
                        cp.start()
                        half_copies[h] = cp

                def _rung(s, o):
                    return pltpu.make_async_remote_copy(
                        src_ref=gather.at[s, :, pl.ds(rs, CA)],
                        dst_ref=gather.at[s, :, pl.ds(rs, CA)],
                        send_sem=rung_send.at[o], recv_sem=rung_recv.at[o],
                        device_id=ynbr,
                        device_id_type=pltpu.DeviceIdType.MESH,
                    )

                rungs = []
                for r in range(ROUNDS):
                    sf = (p - r) % P
                    sb = (p + r) % P
                    if r < 3:
                        fwd_src = gather.at[sf, :, pl.ds(ro, CL)]
                        bwd_src = gather.at[sb, :, pl.ds(ro, CL)]
                    else:
                        fwd_src = gather.at[sf, :, pl.ds(ro, CH)]
                        bwd_src = gather.at[sb, :, pl.ds(ro + CH, CH)]
                    fwd = pltpu.make_async_remote_copy(
                        src_ref=fwd_src, dst_ref=fwd_src,
                        send_sem=fwd_send.at[r], recv_sem=fwd_recv.at[r],
                        device_id=right,
                        device_id_type=pltpu.DeviceIdType.MESH,
                    )
                    fwd.start()
                    bwd = pltpu.make_async_remote_copy(
                        src_ref=bwd_src, dst_ref=bwd_src,
                        send_sem=bwd_send.at[r], recv_sem=bwd_recv.at[r],
                        device_id=left,
                        device_id_type=pltpu.DeviceIdType.MESH,
                    )
                    bwd.start()
                    if r >= 1:
                        rungs.pop(0).wait()
                        rungs.pop(0).wait()
                        _emit(sf)
                        _emit(sb)
                    else:
                        _emit(p)
                    fwd.wait()
                    bwd.wait()
                    if r < 3:
                        for o, s in (
                            (2 * r, (p - 1 - r) % P),
                            (2 * r + 1, (p + 1 + r) % P),
                        ):
                            rg = _rung(s, o)
                            rg.start()
                            rungs.append(rg)
                rg = _rung(anti, 6)
                rg.start()
                rg.wait()
                _emit(anti)
                for h in (0, 1):
                    if half_copies[h] is not None:
                        half_copies[h].wait()


def kernel(dy, W):
    my_x = lax.axis_index("x")
    my_z = lax.axis_index("z")
    p = jnp.where(my_x == 0, my_z, 7 - my_z)
    return pl.pallas_call(
        _body,
        grid_spec=pltpu.PrefetchScalarGridSpec(
            num_scalar_prefetch=1,
            grid=(NQ, K // BK),
            in_specs=[
                pl.BlockSpec(
                    (SM, BK),
                    lambda n, k, pref: (pref[0], jnp.where(n == 0, k, 0)),
                ),
                pl.BlockSpec((WQ, BK), lambda n, k, pref: (n, k)),
            ],
            out_specs=pl.BlockSpec(memory_space=pltpu.MemorySpace.HBM),
            scratch_shapes=[
                pltpu.VMEM((SM, WQ), jnp.float32),
                pltpu.VMEM((SM, K), jnp.bfloat16),
                pltpu.VMEM((P, SM, N), jnp.bfloat16),
                pltpu.VMEM((2, SM, WH), jnp.float32),
                pltpu.SemaphoreType.DMA((NQ,)),
                pltpu.SemaphoreType.DMA((NQ,)),
                pltpu.SemaphoreType.DMA((ROUNDS,)),
                pltpu.SemaphoreType.DMA((ROUNDS,)),
                pltpu.SemaphoreType.DMA((ROUNDS,)),
                pltpu.SemaphoreType.DMA((ROUNDS,)),
                pltpu.SemaphoreType.DMA((7,)),
                pltpu.SemaphoreType.DMA((7,)),
                pltpu.SemaphoreType.DMA((2,)),
            ],
        ),
        out_shape=jax.ShapeDtypeStruct((M, N), jnp.float32),
        compiler_params=pltpu.CompilerParams(
            dimension_semantics=("arbitrary", "arbitrary"),
            has_side_effects=True,
            collective_id=0,
            vmem_limit_bytes=62 * 1024 * 1024,
        ),
    )(p.astype(jnp.int32).reshape(1), dy, W)


# device time: 313979 ns/iter; 1.0937x vs baseline; 1.0937x over previous
import jax
import jax.numpy as jnp
from jax import lax
from jax.experimental import pallas as pl
from jax.experimental.pallas import tpu as pltpu

M = 4096
N = 4096
K = 8192

P = 8
SM = M // P
BK = 256
NQ = 2
WQ = N // NQ
WH = N // 2

ROUNDS = 4
CA = 1280
CO = N - 2 * CA
CL = CA + CO
CH = CL // 2


def _ring_coords(q, my_y):
    xq = q // 4
    zq = jnp.where(xq == 0, q, 7 - q)
    return (xq, my_y, zq)


def _body(
    p_ref,
    dy_ref,
    w_ref,
    out_ref,
    acc,
    gather,
    stage,
    y_send, y_recv,
    fwd_send, fwd_recv,
    bwd_send, bwd_recv,
    rung_send, rung_recv,
    copy_sem,
):
    n = pl.program_id(0)
    k = pl.program_id(1)
    n_last = (n == pl.num_programs(0) - 1) & (k == pl.num_programs(1) - 1)

    my_x = lax.axis_index("x")
    my_y = lax.axis_index("y")
    my_z = lax.axis_index("z")
    p = p_ref[0]
    right = _ring_coords((p + 1) % P, my_y)
    left = _ring_coords((p - 1) % P, my_y)
    ynbr = (my_x, 1 - my_y, my_z)
    anti = (p + 4) % P

    @pl.when((n == 0) & (k == 0))
    def _():
        barrier = pltpu.get_barrier_semaphore()
        for nbr in (ynbr, left, right):
            pl.semaphore_signal(
                barrier, inc=1, device_id=nbr,
                device_id_type=pltpu.DeviceIdType.MESH,
            )
        pl.semaphore_wait(barrier, 3)

    @pl.when(k == 0)
    def _():
        acc[...] = jnp.zeros_like(acc)

    a = dy_ref[...].astype(jnp.bfloat16)
    b = w_ref[...].astype(jnp.bfloat16)
    acc[...] += lax.dot_general(
        a, b, (((1,), (1,)), ((), ())), preferred_element_type=jnp.float32
    )

    def _y_quarter(h):
        return pltpu.make_async_remote_copy(
            src_ref=gather.at[p, :, pl.ds(h * WQ, WQ)],
            dst_ref=gather.at[anti, :, pl.ds(h * WQ, WQ)],
            send_sem=y_send.at[h], recv_sem=y_recv.at[h],
            device_id=ynbr, device_id_type=pltpu.DeviceIdType.MESH,
        )

    for h in range(NQ):
        @pl.when((n == h) & (k == pl.num_programs(1) - 1))
        def _(h=h):
            gather[p, :, pl.ds(h * WQ, WQ)] = acc[...].astype(jnp.bfloat16)
            _y_quarter(h).start()

    @pl.when(n_last)
    def _():
        for h in range(NQ):
            _y_quarter(h).wait()
        gather[p] = (
            gather[p].astype(jnp.float32) + gather[anti].astype(jnp.float32)
        ).astype(jnp.bfloat16)

        for yv in (0, 1):

            @pl.when(my_y == yv)
            def _(yv=yv):
                ro = CA * yv
                rs = CL * yv

                half_copies = [None, None]

                def _emit(s):
                    for h in (0, 1):
                        if half_copies[h] is not None:
                            half_copies[h].wait()
                        stage[h] = gather[
                            s, :, h * WH:(h + 1) * WH
                        ].astype(jnp.float32)
                        cp = pltpu.make_async_copy(
                            stage.at[h],
                            out_ref.at[pl.ds(s * SM, SM), pl.ds(h * WH, WH)],
                            copy_sem.at[h],
                        )
                        cp.start()
                        half_copies[h] = cp

                def _rung(s, o):
                    return pltpu.make_async_remote_copy(
                        src_ref=gather.at[s, :, pl.ds(rs, CA)],
                        dst_ref=gather.at[s, :, pl.ds(rs, CA)],
                        send_sem=rung_send.at[o], recv_sem=rung_recv.at[o],
                        device_id=ynbr,
                        device_id_type=pltpu.DeviceIdType.MESH,
                    )

                rungs = []
                for r in range(ROUNDS):
                    sf = (p - r) % P
                    sb = (p + r) % P
                    if r < 3:
                        fwd_src = gather.at[sf, :, pl.ds(ro, CL)]
                        bwd_src = gather.at[sb, :, pl.ds(ro, CL)]
                    else:
                        fwd_src = gather.at[sf, :, pl.ds(ro, CH)]
                        bwd_src = gather.at[sb, :, pl.ds(ro + CH, CH)]
                    fwd = pltpu.make_async_remote_copy(
                        src_ref=fwd_src, dst_ref=fwd_src,
                        send_sem=fwd_send.at[r], recv_sem=fwd_recv.at[r],
                        device_id=right,
                        device_id_type=pltpu.DeviceIdType.MESH,
                    )
                    fwd.start()
                    bwd = pltpu.make_async_remote_copy(
                        src_ref=bwd_src, dst_ref=bwd_src,
                        send_sem=bwd_send.at[r], recv_sem=bwd_recv.at[r],
                        device_id=left,
                        device_id_type=pltpu.DeviceIdType.MESH,
                    )
                    bwd.start()
                    if r >= 1:
                        rungs.pop(0).wait()
                        rungs.pop(0).wait()
                        _emit(sf)
                        _emit(sb)
                    else:
                        _emit(p)
                    fwd.wait()
                    bwd.wait()
                    if r < 3:
                        for o, s in (
                            (2 * r, (p - 1 - r) % P),
                            (2 * r + 1, (p + 1 + r) % P),
                        ):
                            rg = _rung(s, o)
                            rg.start()
                            rungs.append(rg)
                rg = _rung(anti, 6)
                rg.start()
                rg.wait()
                _emit(anti)
                for h in (0, 1):
                    if half_copies[h] is not None:
                        half_copies[h].wait()


def kernel(dy, W):
    my_x = lax.axis_index("x")
    my_z = lax.axis_index("z")
    p = jnp.where(my_x == 0, my_z, 7 - my_z)
    return pl.pallas_call(
        _body,
        grid_spec=pltpu.PrefetchScalarGridSpec(
            num_scalar_prefetch=1,
            grid=(NQ, K // BK),
            in_specs=[
                pl.BlockSpec((SM, BK), lambda n, k, pref: (pref[0], k)),
                pl.BlockSpec((WQ, BK), lambda n, k, pref: (n, k)),
            ],
            out_specs=pl.BlockSpec(memory_space=pltpu.MemorySpace.HBM),
            scratch_shapes=[
                pltpu.VMEM((SM, WQ), jnp.float32),
                pltpu.VMEM((P, SM, N), jnp.bfloat16),
                pltpu.VMEM((2, SM, WH), jnp.float32),
                pltpu.SemaphoreType.DMA((NQ,)),
                pltpu.SemaphoreType.DMA((NQ,)),
                pltpu.SemaphoreType.DMA((ROUNDS,)),
                pltpu.SemaphoreType.DMA((ROUNDS,)),
                pltpu.SemaphoreType.DMA((ROUNDS,)),
                pltpu.SemaphoreType.DMA((ROUNDS,)),
                pltpu.SemaphoreType.DMA((7,)),
                pltpu.SemaphoreType.DMA((7,)),
                pltpu.SemaphoreType.DMA((2,)),
            ],
        ),
        out_shape=jax.ShapeDtypeStruct((M, N), jnp.float32),
        compiler_params=pltpu.CompilerParams(
            dimension_semantics=("arbitrary", "arbitrary"),
            has_side_effects=True,
            collective_id=0,
            vmem_limit_bytes=62 * 1024 * 1024,
        ),
    )(p.astype(jnp.int32).reshape(1), dy, W)


# device time: 294202 ns/iter; 1.1672x vs baseline; 1.0672x over previous
import jax
import jax.numpy as jnp
from jax import lax
from jax.experimental import pallas as pl
from jax.experimental.pallas import tpu as pltpu

M = 4096
N = 4096
K = 8192

P = 8
SM = M // P
BK = 512
NQ = 2
WQ = N // NQ
WH = N // 2

ROUNDS = 4
CA = 1280
CO = N - 2 * CA
CL = CA + CO
CH = CL // 2


def _ring_coords(q, my_y):
    xq = q // 4
    zq = jnp.where(xq == 0, q, 7 - q)
    return (xq, my_y, zq)


def _body(
    p_ref,
    dy_ref,
    w_ref,
    out_ref,
    acc,
    gather,
    stage,
    y_send, y_recv,
    fwd_send, fwd_recv,
    bwd_send, bwd_recv,
    rung_send, rung_recv,
    copy_sem,
):
    n = pl.program_id(0)
    k = pl.program_id(1)
    n_last = (n == pl.num_programs(0) - 1) & (k == pl.num_programs(1) - 1)

    my_x = lax.axis_index("x")
    my_y = lax.axis_index("y")
    my_z = lax.axis_index("z")
    p = p_ref[0]
    right = _ring_coords((p + 1) % P, my_y)
    left = _ring_coords((p - 1) % P, my_y)
    ynbr = (my_x, 1 - my_y, my_z)
    anti = (p + 4) % P

    @pl.when((n == 0) & (k == 0))
    def _():
        barrier = pltpu.get_barrier_semaphore()
        for nbr in (ynbr, left, right):
            pl.semaphore_signal(
                barrier, inc=1, device_id=nbr,
                device_id_type=pltpu.DeviceIdType.MESH,
            )
        pl.semaphore_wait(barrier, 3)

    @pl.when(k == 0)
    def _():
        acc[...] = jnp.zeros_like(acc)

    a = dy_ref[...].astype(jnp.bfloat16)
    b = w_ref[...].astype(jnp.bfloat16)
    acc[...] += lax.dot_general(
        a, b, (((1,), (1,)), ((), ())), preferred_element_type=jnp.float32
    )

    def _y_quarter(h):
        return pltpu.make_async_remote_copy(
            src_ref=gather.at[p, :, pl.ds(h * WQ, WQ)],
            dst_ref=gather.at[anti, :, pl.ds(h * WQ, WQ)],
            send_sem=y_send.at[h], recv_sem=y_recv.at[h],
            device_id=ynbr, device_id_type=pltpu.DeviceIdType.MESH,
        )

    for h in range(NQ):
        @pl.when((n == h) & (k == pl.num_programs(1) - 1))
        def _(h=h):
            gather[p, :, pl.ds(h * WQ, WQ)] = acc[...].astype(jnp.bfloat16)
            _y_quarter(h).start()

    @pl.when(n_last)
    def _():
        for h in range(NQ):
            _y_quarter(h).wait()
        gather[p] = (
            gather[p].astype(jnp.float32) + gather[anti].astype(jnp.float32)
        ).astype(jnp.bfloat16)

        for yv in (0, 1):

            @pl.when(my_y == yv)
            def _(yv=yv):
                ro = CA * yv
                rs = CL * yv

                half_copies = [None, None]

                def _emit(s):
                    for h in (0, 1):
                        if half_copies[h] is not None:
                            half_copies[h].wait()
                        stage[h] = gather[
                            s, :, h * WH:(h + 1) * WH
                        ].astype(jnp.float32)
                        cp = pltpu.make_async_copy(
                            stage.at[h],
                            out_ref.at[pl.ds(s * SM, SM), pl.ds(h * WH, WH)],
                            copy_sem.at[h],
                        )
                        cp.start()
                        half_copies[h] = cp

                def _rung(s, o):
                    return pltpu.make_async_remote_copy(
                        src_ref=gather.at[s, :, pl.ds(rs, CA)],
                        dst_ref=gather.at[s, :, pl.ds(rs, CA)],
                        send_sem=rung_send.at[o], recv_sem=rung_recv.at[o],
                        device_id=ynbr,
                        device_id_type=pltpu.DeviceIdType.MESH,
                    )

                rungs = []
                for r in range(ROUNDS):
                    sf = (p - r) % P
                    sb = (p + r) % P
                    if r < 3:
                        fwd_src = gather.at[sf, :, pl.ds(ro, CL)]
                        bwd_src = gather.at[sb, :, pl.ds(ro, CL)]
                    else:
                        fwd_src = gather.at[sf, :, pl.ds(ro, CH)]
                        bwd_src = gather.at[sb, :, pl.ds(ro + CH, CH)]
                    fwd = pltpu.make_async_remote_copy(
                        src_ref=fwd_src, dst_ref=fwd_src,
                        send_sem=fwd_send.at[r], recv_sem=fwd_recv.at[r],
                        device_id=right,
                        device_id_type=pltpu.DeviceIdType.MESH,
                    )
                    fwd.start()
                    bwd = pltpu.make_async_remote_copy(
                        src_ref=bwd_src, dst_ref=bwd_src,
                        send_sem=bwd_send.at[r], recv_sem=bwd_recv.at[r],
                        device_id=left,
                        device_id_type=pltpu.DeviceIdType.MESH,
                    )
                    bwd.start()
                    if r >= 1:
                        rungs.pop(0).wait()
                        rungs.pop(0).wait()
                        _emit(sf)
                        _emit(sb)
                    else:
                        _emit(p)
                    fwd.wait()
                    bwd.wait()
                    if r < 3:
                        for o, s in (
                            (2 * r, (p - 1 - r) % P),
                            (2 * r + 1, (p + 1 + r) % P),
                        ):
                            rg = _rung(s, o)
                            rg.start()
                            rungs.append(rg)
                rg = _rung(anti, 6)
                rg.start()
                rg.wait()
                _emit(anti)
                for h in (0, 1):
                    if half_copies[h] is not None:
                        half_copies[h].wait()


def kernel(dy, W):
    my_x = lax.axis_index("x")
    my_z = lax.axis_index("z")
    p = jnp.where(my_x == 0, my_z, 7 - my_z)
    return pl.pallas_call(
        _body,
        grid_spec=pltpu.PrefetchScalarGridSpec(
            num_scalar_prefetch=1,
            grid=(NQ, K // BK),
            in_specs=[
                pl.BlockSpec((SM, BK), lambda n, k, pref: (pref[0], k)),
                pl.BlockSpec((WQ, BK), lambda n, k, pref: (n, k)),
            ],
            out_specs=pl.BlockSpec(memory_space=pltpu.MemorySpace.HBM),
            scratch_shapes=[
                pltpu.VMEM((SM, WQ), jnp.float32),
                pltpu.VMEM((P, SM, N), jnp.bfloat16),
                pltpu.VMEM((2, SM, WH), jnp.float32),
                pltpu.SemaphoreType.DMA((NQ,)),
                pltpu.SemaphoreType.DMA((NQ,)),
                pltpu.SemaphoreType.DMA((ROUNDS,)),
                pltpu.SemaphoreType.DMA((ROUNDS,)),
                pltpu.SemaphoreType.DMA((ROUNDS,)),
                pltpu.SemaphoreType.DMA((ROUNDS,)),
                pltpu.SemaphoreType.DMA((7,)),
                pltpu.SemaphoreType.DMA((7,)),
                pltpu.SemaphoreType.DMA((2,)),
            ],
        ),
        out_shape=jax.ShapeDtypeStruct((M, N), jnp.float32),
        compiler_params=pltpu.CompilerParams(
            dimension_semantics=("arbitrary", "arbitrary"),
            has_side_effects=True,
            collective_id=0,
            vmem_limit_bytes=62 * 1024 * 1024,
        ),
    )(p.astype(jnp.int32).reshape(1), dy, W)
